# Initial kernel scaffold; baseline (speedup 1.0000x reference)
import jax
import jax.numpy as jnp
from jax.experimental import pallas as pl


def kernel(x, batch, params):
    raise NotImplementedError("write your pallas kernel here")


# speedup vs baseline: 1.9383x; 1.9383x over previous
import functools

import jax
import jax.numpy as jnp
from jax.experimental import pallas as pl
from jax.experimental.pallas import tpu as pltpu

N = 8192
K = 20
NB = 4
BLK = 256
BIG = 1e30
BIG2 = 2e30


def _bn(h, g, be):
    m = jnp.mean(h, axis=0, keepdims=True)
    v = jnp.mean(h * h, axis=0, keepdims=True) - m * m
    return g * (h - m) / jnp.sqrt(v + 1e-5) + be


def _fc1_body(x_ref, w1_ref, b1_ref, g1_ref, be1_ref, w2_ref, b2_ref,
              g2_ref, be2_ref, h1_ref):
    x = x_ref[...]
    h = jnp.dot(x, w1_ref[...], preferred_element_type=jnp.float32, precision=jax.lax.Precision.HIGHEST)
    h = jnp.maximum(h + b1_ref[...], 0.0)
    h = _bn(h, g1_ref[...], be1_ref[...])
    h = jnp.dot(h, w2_ref[...], preferred_element_type=jnp.float32,
                precision=jax.lax.Precision.HIGHEST)
    h = jnp.maximum(h + b2_ref[...], 0.0)
    h = _bn(h, g2_ref[...], be2_ref[...])
    h1_ref[...] = h


def _knn_body(h1_ref, batch_ref, mean_ref, max_ref, d_ref):
    i = pl.program_id(0)
    h1 = h1_ref[...]
    hblk = h1_ref[pl.ds(i * BLK, BLK), :]
    sq = jnp.sum(h1 * h1, axis=1)[None, :]
    sqb = jnp.sum(hblk * hblk, axis=1)[:, None]
    d = sqb + sq - 2.0 * jax.lax.dot_general(
        hblk, h1, (((1,), (1,)), ((), ())),
        preferred_element_type=jnp.float32)
    bj = batch_ref[...]
    bi = batch_ref[0, pl.ds(i * BLK, BLK)][:, None]
    jcol = jax.lax.broadcasted_iota(jnp.int32, (BLK, N), 1)
    rowg = i * BLK + jax.lax.broadcasted_iota(jnp.int32, (BLK, N), 0)
    d_ref[...] = jnp.where((bi != bj) | (jcol == rowg), BIG, d)

    def body(_, carry):
        summ, maxx = carry
        d = d_ref[...]
        jcol = jax.lax.broadcasted_iota(jnp.int32, (BLK, N), 1)
        m = jnp.min(d, axis=1, keepdims=True)
        jidx = jnp.min(jnp.where(d == m, jcol, N), axis=1,
                       keepdims=True)
        sel = jcol == jidx
        rows = jnp.dot(sel.astype(jnp.float32), h1,
                       preferred_element_type=jnp.float32,
                       precision=jax.lax.Precision.HIGHEST)
        d_ref[...] = jnp.where(sel, BIG2, d)
        return summ + rows, jnp.maximum(maxx, rows)

    summ, maxx = jax.lax.fori_loop(
        0, K, body,
        (jnp.zeros((BLK, 128), jnp.float32),
         jnp.full((BLK, 128), -jnp.inf, jnp.float32)))
    mean_ref[...] = summ * (1.0 / K)
    max_ref[...] = maxx


def _head_body(mean_ref, max_ref, batch_ref, eps_ref,
               mw1_ref, mb1_ref, mg1_ref, mbe1_ref,
               mw2_ref, mb2_ref, mg2_ref, mbe2_ref,
               sw1_ref, sb1_ref, sg1_ref, sbe1_ref,
               sw2_ref, sb2_ref, sg2_ref, sbe2_ref,
               w3_ref, b3_ref, w4_ref, b4_ref,
               rec_ref, mu_ref, logvar_ref):
    def mlp(h, w1, b1, g1, be1, w2, b2, g2, be2):
        h = jnp.dot(h, w1[...], preferred_element_type=jnp.float32)
        h = jnp.maximum(h + b1[...], 0.0)
        h = _bn(h, g1[...], be1[...])
        h = jnp.dot(h, w2[...], preferred_element_type=jnp.float32)
        h = jnp.maximum(h + b2[...], 0.0)
        h = _bn(h, g2[...], be2[...])
        return h

    mu_n = mlp(mean_ref[...], mw1_ref, mb1_ref, mg1_ref, mbe1_ref,
               mw2_ref, mb2_ref, mg2_ref, mbe2_ref)
    sig_n = mlp(max_ref[...], sw1_ref, sb1_ref, sg1_ref, sbe1_ref,
                sw2_ref, sb2_ref, sg2_ref, sbe2_ref)

    b = batch_ref[...]
    seg = jax.lax.broadcasted_iota(jnp.int32, (NB, N), 0)
    onehot = (seg == b).astype(jnp.float32)
    cnt = jnp.sum(onehot, axis=1, keepdims=True)
    inv = 1.0 / jnp.maximum(cnt, 1.0)
    mu = jnp.dot(onehot, mu_n, preferred_element_type=jnp.float32, precision=jax.lax.Precision.HIGHEST) * inv
    logvar = jnp.dot(onehot, sig_n, preferred_element_type=jnp.float32, precision=jax.lax.Precision.HIGHEST) * inv
    z = mu + eps_ref[...] * jnp.exp(0.5 * logvar)
    h3 = jnp.maximum(
        jnp.dot(z, w3_ref[...], preferred_element_type=jnp.float32)
        + b3_ref[...], 0.0)
    rec = jnp.dot(h3, w4_ref[...], preferred_element_type=jnp.float32) \
        + b4_ref[...]
    rec_ref[...] = rec
    mu_ref[...] = mu
    logvar_ref[...] = logvar


def _bn_ref(h, g, be):
    m = jnp.mean(h, axis=0)
    v = jnp.var(h, axis=0)
    return g * (h - m) / jnp.sqrt(v + 1e-5) + be


def kernel(x, batch, params):
    p = params
    batch2 = batch.reshape(1, N)

    h = x
    for i in (1, 2):
        h = h @ p['fc1_W%d' % i] + p['fc1_b%d' % i]
        h = jax.nn.relu(h)
        h = _bn_ref(h, p['fc1_g%d' % i], p['fc1_be%d' % i])
    h1 = h

    nbr_mean, nbr_max = pl.pallas_call(
        _knn_body,
        grid=(N // BLK,),
        in_specs=[
            pl.BlockSpec((N, 128), lambda i: (0, 0)),
            pl.BlockSpec((1, N), lambda i: (0, 0)),
        ],
        out_specs=[
            pl.BlockSpec((BLK, 128), lambda i: (i, 0)),
            pl.BlockSpec((BLK, 128), lambda i: (i, 0)),
        ],
        out_shape=[
            jax.ShapeDtypeStruct((N, 128), jnp.float32),
            jax.ShapeDtypeStruct((N, 128), jnp.float32),
        ],
        scratch_shapes=[pltpu.VMEM((BLK, N), jnp.float32)],
    )(h1, batch2)

    eps = jax.random.normal(jax.random.key(1), (NB, 128), dtype=jnp.float32)

    rec, mu, logvar = pl.pallas_call(
        _head_body,
        out_shape=[
            jax.ShapeDtypeStruct((NB, 3 * 2048), jnp.float32),
            jax.ShapeDtypeStruct((NB, 128), jnp.float32),
            jax.ShapeDtypeStruct((NB, 128), jnp.float32),
        ],
    )(nbr_mean, nbr_max, batch2, eps,
      p['mu_W1'], p['mu_b1'][None, :], p['mu_g1'][None, :],
      p['mu_be1'][None, :], p['mu_W2'], p['mu_b2'][None, :],
      p['mu_g2'][None, :], p['mu_be2'][None, :],
      p['sig_W1'], p['sig_b1'][None, :], p['sig_g1'][None, :],
      p['sig_be1'][None, :], p['sig_W2'], p['sig_b2'][None, :],
      p['sig_g2'][None, :], p['sig_be2'][None, :],
      p['fc3_W'], p['fc3_b'][None, :], p['fc4_W'], p['fc4_b'][None, :])

    return (rec, mu, logvar)


# speedup vs baseline: 6.2102x; 3.2040x over previous
import functools

import jax
import jax.numpy as jnp
from jax import lax
from jax.experimental import pallas as pl
from jax.experimental.pallas import tpu as pltpu
from jax.experimental.pallas import tpu_sc as plsc

N = 8192
K = 20
NB = 4
BLK = 256
BIG = 1e30
BIG2 = 2e30


def _bn(h, g, be):
    m = jnp.mean(h, axis=0, keepdims=True)
    v = jnp.mean(h * h, axis=0, keepdims=True) - m * m
    return g * (h - m) / jnp.sqrt(v + 1e-5) + be


def _knn_idx_body(h1_ref, batch_ref, idx_ref, d_ref):
    i = pl.program_id(0)
    h1 = h1_ref[...]
    hblk = h1_ref[pl.ds(i * BLK, BLK), :]
    sq = jnp.sum(h1 * h1, axis=1)[None, :]
    sqb = jnp.sum(hblk * hblk, axis=1)[:, None]
    d = sqb + sq - 2.0 * jax.lax.dot_general(
        hblk, h1, (((1,), (1,)), ((), ())),
        preferred_element_type=jnp.float32)
    bj = batch_ref[...]
    bi = batch_ref[0, pl.ds(i * BLK, BLK)][:, None]
    jcol = jax.lax.broadcasted_iota(jnp.int32, (BLK, N), 1)
    rowg = i * BLK + jax.lax.broadcasted_iota(jnp.int32, (BLK, N), 0)
    d_ref[...] = jnp.where((bi != bj) | (jcol == rowg), BIG, d)

    kcol = jax.lax.broadcasted_iota(jnp.int32, (BLK, K), 1)

    def body(t, idx_acc):
        d = d_ref[...]
        jcol = jax.lax.broadcasted_iota(jnp.int32, (BLK, N), 1)
        m = jnp.min(d, axis=1, keepdims=True)
        jidx = jnp.min(jnp.where(d == m, jcol, N), axis=1,
                       keepdims=True)
        sel = jcol == jidx
        d_ref[...] = jnp.where(sel, BIG2, d)
        return jnp.where(kcol == t, jidx, idx_acc)

    idx_ref[...] = jax.lax.fori_loop(
        0, K, body, jnp.zeros((BLK, K), jnp.int32))


def _sc_gather_meanmax(h1, idx_flat):
    info = plsc.get_sparse_core_info()
    nc, ns = info.num_cores, info.num_subcores
    nw = nc * ns
    rpw = N // nw
    ch = 8
    hf = ch // 2
    nch = rpw // ch
    mesh = plsc.VectorSubcoreMesh(core_axis_name="c", subcore_axis_name="s")

    @functools.partial(
        pl.kernel, mesh=mesh,
        out_type=[
            jax.ShapeDtypeStruct((N, 128), jnp.float32),
            jax.ShapeDtypeStruct((N, 128), jnp.float32),
        ],
        scratch_types=[
            pltpu.VMEM((rpw * K,), jnp.int32),
            pltpu.VMEM((8 * K, 128), jnp.float32),
            pltpu.VMEM((8, 128), jnp.float32),
            pltpu.VMEM((8, 128), jnp.float32),
            pltpu.SemaphoreType.DMA,
        ],
    )
    def k(h1_hbm, idx_hbm, mean_hbm, max_hbm, idxv, rows, mo, mx, sem):
        wid = lax.axis_index("s") * nc + lax.axis_index("c")
        base = wid * rpw
        pltpu.sync_copy(idx_hbm.at[pl.ds(base * K, rpw * K)], idxv)

        def chunk(c, carry):
            c0 = c * ch
            cp1 = pltpu.async_copy(
                h1_hbm.at[idxv.at[pl.ds(c0 * K, hf * K)]],
                rows.at[pl.ds(0, hf * K)], sem)
            cp2 = pltpu.async_copy(
                h1_hbm.at[idxv.at[pl.ds(c0 * K + hf * K, hf * K)]],
                rows.at[pl.ds(hf * K, hf * K)], sem)
            cp1.wait()
            cp2.wait()
            for r in range(ch):
                for l in range(8):
                    sl = pl.ds(l * 16, 16)
                    s = rows[r * K, sl]
                    m = rows[r * K, sl]
                    for j in range(1, K):
                        v = rows[r * K + j, sl]
                        s = s + v
                        m = jnp.maximum(m, v)
                    mo[r, sl] = s * (1.0 / K)
                    mx[r, sl] = m
            pltpu.sync_copy(mo, mean_hbm.at[pl.ds(base + c0, ch)])
            pltpu.sync_copy(mx, max_hbm.at[pl.ds(base + c0, ch)])
            return carry

        lax.fori_loop(0, nch, chunk, 0)

    return k(h1, idx_flat)


def _head_body(mean_ref, max_ref, batch_ref, eps_ref,
               mw1_ref, mb1_ref, mg1_ref, mbe1_ref,
               mw2_ref, mb2_ref, mg2_ref, mbe2_ref,
               sw1_ref, sb1_ref, sg1_ref, sbe1_ref,
               sw2_ref, sb2_ref, sg2_ref, sbe2_ref,
               w3_ref, b3_ref, w4_ref, b4_ref,
               rec_ref, mu_ref, logvar_ref):
    def mlp(h, w1, b1, g1, be1, w2, b2, g2, be2):
        h = jnp.dot(h, w1[...], preferred_element_type=jnp.float32)
        h = jnp.maximum(h + b1[...], 0.0)
        h = _bn(h, g1[...], be1[...])
        h = jnp.dot(h, w2[...], preferred_element_type=jnp.float32)
        h = jnp.maximum(h + b2[...], 0.0)
        h = _bn(h, g2[...], be2[...])
        return h

    mu_n = mlp(mean_ref[...], mw1_ref, mb1_ref, mg1_ref, mbe1_ref,
               mw2_ref, mb2_ref, mg2_ref, mbe2_ref)
    sig_n = mlp(max_ref[...], sw1_ref, sb1_ref, sg1_ref, sbe1_ref,
                sw2_ref, sb2_ref, sg2_ref, sbe2_ref)

    b = batch_ref[...]
    seg = jax.lax.broadcasted_iota(jnp.int32, (NB, N), 0)
    onehot = (seg == b).astype(jnp.float32)
    cnt = jnp.sum(onehot, axis=1, keepdims=True)
    inv = 1.0 / jnp.maximum(cnt, 1.0)
    mu = jnp.dot(onehot, mu_n, preferred_element_type=jnp.float32,
                 precision=jax.lax.Precision.HIGHEST) * inv
    logvar = jnp.dot(onehot, sig_n, preferred_element_type=jnp.float32,
                     precision=jax.lax.Precision.HIGHEST) * inv
    z = mu + eps_ref[...] * jnp.exp(0.5 * logvar)
    h3 = jnp.maximum(
        jnp.dot(z, w3_ref[...], preferred_element_type=jnp.float32)
        + b3_ref[...], 0.0)
    rec = jnp.dot(h3, w4_ref[...], preferred_element_type=jnp.float32) \
        + b4_ref[...]
    rec_ref[...] = rec
    mu_ref[...] = mu
    logvar_ref[...] = logvar


def _bn_ref(h, g, be):
    m = jnp.mean(h, axis=0)
    v = jnp.var(h, axis=0)
    return g * (h - m) / jnp.sqrt(v + 1e-5) + be


def kernel(x, batch, params):
    p = params
    batch2 = batch.reshape(1, N)

    h = x
    for i in (1, 2):
        h = h @ p['fc1_W%d' % i] + p['fc1_b%d' % i]
        h = jax.nn.relu(h)
        h = _bn_ref(h, p['fc1_g%d' % i], p['fc1_be%d' % i])
    h1 = h

    idx = pl.pallas_call(
        _knn_idx_body,
        grid=(N // BLK,),
        in_specs=[
            pl.BlockSpec((N, 128), lambda i: (0, 0)),
            pl.BlockSpec((1, N), lambda i: (0, 0)),
        ],
        out_specs=pl.BlockSpec((BLK, K), lambda i: (i, 0)),
        out_shape=jax.ShapeDtypeStruct((N, K), jnp.int32),
        scratch_shapes=[pltpu.VMEM((BLK, N), jnp.float32)],
    )(h1, batch2)

    nbr_mean, nbr_max = _sc_gather_meanmax(h1, idx.reshape(N * K))

    eps = jax.random.normal(jax.random.key(1), (NB, 128), dtype=jnp.float32)

    rec, mu, logvar = pl.pallas_call(
        _head_body,
        out_shape=[
            jax.ShapeDtypeStruct((NB, 3 * 2048), jnp.float32),
            jax.ShapeDtypeStruct((NB, 128), jnp.float32),
            jax.ShapeDtypeStruct((NB, 128), jnp.float32),
        ],
    )(nbr_mean, nbr_max, batch2, eps,
      p['mu_W1'], p['mu_b1'][None, :], p['mu_g1'][None, :],
      p['mu_be1'][None, :], p['mu_W2'], p['mu_b2'][None, :],
      p['mu_g2'][None, :], p['mu_be2'][None, :],
      p['sig_W1'], p['sig_b1'][None, :], p['sig_g1'][None, :],
      p['sig_be1'][None, :], p['sig_W2'], p['sig_b2'][None, :],
      p['sig_g2'][None, :], p['sig_be2'][None, :],
      p['fc3_W'], p['fc3_b'][None, :], p['fc4_W'], p['fc4_b'][None, :])

    return (rec, mu, logvar)


# speedup vs baseline: 7.7207x; 1.2432x over previous
import functools

import jax
import jax.numpy as jnp
from jax import lax
from jax.experimental import pallas as pl
from jax.experimental.pallas import tpu as pltpu
from jax.experimental.pallas import tpu_sc as plsc

N = 8192
K = 20
NB = 4
BLK = 256
BIG = 1e30
BIG2 = 2e30


def _bn(h, g, be):
    m = jnp.mean(h, axis=0, keepdims=True)
    v = jnp.mean(h * h, axis=0, keepdims=True) - m * m
    return g * (h - m) / jnp.sqrt(v + 1e-5) + be


CW = 512


def _knn_idx_body(h1_ref, batch_ref, idx_ref, d_ref):
    i = pl.program_id(0)
    hblk = h1_ref[pl.ds(i * BLK, BLK), :]
    sqb = jnp.sum(hblk * hblk, axis=1)[:, None]
    bj_row = batch_ref[...]
    bi = batch_ref[0, pl.ds(i * BLK, BLK)][:, None]
    jrow = jax.lax.broadcasted_iota(jnp.int32, (1, N), 1)

    b_first = jnp.min(bi)
    b_last = jnp.max(bi)
    j0 = jnp.min(jnp.where(bj_row == b_first, jrow, N))
    j1 = jnp.max(jnp.where(bj_row == b_last, jrow, -1)) + 1
    c_lo = j0 // CW
    c_hi = (j1 + CW - 1) // CW

    def compute_chunk(c, carry):
        h1c = h1_ref[pl.ds(c * CW, CW), :]
        sqc = jnp.sum(h1c * h1c, axis=1)[None, :]
        dc = sqb + sqc - 2.0 * jax.lax.dot_general(
            hblk, h1c, (((1,), (1,)), ((), ())),
            preferred_element_type=jnp.float32)
        bjc = batch_ref[0, pl.ds(c * CW, CW)][None, :]
        jcolc = c * CW + jax.lax.broadcasted_iota(jnp.int32, (BLK, CW), 1)
        rowg = i * BLK + jax.lax.broadcasted_iota(jnp.int32, (BLK, CW), 0)
        d_ref[:, pl.ds(c * CW, CW)] = jnp.where(
            (bi != bjc) | (jcolc == rowg), BIG, dc)
        return carry

    jax.lax.fori_loop(c_lo, c_hi, compute_chunk, 0)

    kcol = jax.lax.broadcasted_iota(jnp.int32, (BLK, K), 1)

    def body(t, idx_acc):
        def scan_chunk(c, carry):
            m, jidx = carry
            dc = d_ref[:, pl.ds(c * CW, CW)]
            jcolc = c * CW + jax.lax.broadcasted_iota(
                jnp.int32, (BLK, CW), 1)
            mc = jnp.min(dc, axis=1, keepdims=True)
            jc = jnp.min(jnp.where(dc == mc, jcolc, N), axis=1,
                         keepdims=True)
            jidx = jnp.where(mc < m, jc,
                             jnp.where(mc == m, jnp.minimum(jidx, jc),
                                       jidx))
            return jnp.minimum(m, mc), jidx

        m0 = jnp.full((BLK, 1), BIG2, jnp.float32)
        i0 = jnp.full((BLK, 1), N, jnp.int32)
        m, jidx = jax.lax.fori_loop(c_lo, c_hi, scan_chunk, (m0, i0))

        def clear_chunk(c, carry):
            dc = d_ref[:, pl.ds(c * CW, CW)]
            jcolc = c * CW + jax.lax.broadcasted_iota(
                jnp.int32, (BLK, CW), 1)
            d_ref[:, pl.ds(c * CW, CW)] = jnp.where(
                jcolc == jidx, BIG2, dc)
            return carry

        jax.lax.fori_loop(c_lo, c_hi, clear_chunk, 0)
        return jnp.where(kcol == t, jidx, idx_acc)

    idx_ref[...] = jax.lax.fori_loop(
        0, K, body, jnp.zeros((BLK, K), jnp.int32))


def _sc_gather_meanmax(h1, idx_flat):
    info = plsc.get_sparse_core_info()
    nc, ns = info.num_cores, info.num_subcores
    nw = nc * ns
    rpw = N // nw
    ch = 8
    hf = ch // 2
    nch = rpw // ch
    mesh = plsc.VectorSubcoreMesh(core_axis_name="c", subcore_axis_name="s")

    @functools.partial(
        pl.kernel, mesh=mesh,
        out_type=[
            jax.ShapeDtypeStruct((N, 128), jnp.float32),
            jax.ShapeDtypeStruct((N, 128), jnp.float32),
        ],
        scratch_types=[
            pltpu.VMEM((rpw * K,), jnp.int32),
            pltpu.VMEM((8 * K, 128), jnp.float32),
            pltpu.VMEM((8, 128), jnp.float32),
            pltpu.VMEM((8, 128), jnp.float32),
            pltpu.SemaphoreType.DMA,
        ],
    )
    def k(h1_hbm, idx_hbm, mean_hbm, max_hbm, idxv, rows, mo, mx, sem):
        wid = lax.axis_index("s") * nc + lax.axis_index("c")
        base = wid * rpw
        pltpu.sync_copy(idx_hbm.at[pl.ds(base * K, rpw * K)], idxv)

        def chunk(c, carry):
            c0 = c * ch
            cp1 = pltpu.async_copy(
                h1_hbm.at[idxv.at[pl.ds(c0 * K, hf * K)]],
                rows.at[pl.ds(0, hf * K)], sem)
            cp2 = pltpu.async_copy(
                h1_hbm.at[idxv.at[pl.ds(c0 * K + hf * K, hf * K)]],
                rows.at[pl.ds(hf * K, hf * K)], sem)
            cp1.wait()
            cp2.wait()
            for r in range(ch):
                for l in range(8):
                    sl = pl.ds(l * 16, 16)
                    s = rows[r * K, sl]
                    m = rows[r * K, sl]
                    for j in range(1, K):
                        v = rows[r * K + j, sl]
                        s = s + v
                        m = jnp.maximum(m, v)
                    mo[r, sl] = s * (1.0 / K)
                    mx[r, sl] = m
            pltpu.sync_copy(mo, mean_hbm.at[pl.ds(base + c0, ch)])
            pltpu.sync_copy(mx, max_hbm.at[pl.ds(base + c0, ch)])
            return carry

        lax.fori_loop(0, nch, chunk, 0)

    return k(h1, idx_flat)


def _head_body(mean_ref, max_ref, batch_ref, eps_ref,
               mw1_ref, mb1_ref, mg1_ref, mbe1_ref,
               mw2_ref, mb2_ref, mg2_ref, mbe2_ref,
               sw1_ref, sb1_ref, sg1_ref, sbe1_ref,
               sw2_ref, sb2_ref, sg2_ref, sbe2_ref,
               w3_ref, b3_ref, w4_ref, b4_ref,
               rec_ref, mu_ref, logvar_ref):
    def mlp(h, w1, b1, g1, be1, w2, b2, g2, be2):
        h = jnp.dot(h, w1[...], preferred_element_type=jnp.float32)
        h = jnp.maximum(h + b1[...], 0.0)
        h = _bn(h, g1[...], be1[...])
        h = jnp.dot(h, w2[...], preferred_element_type=jnp.float32)
        h = jnp.maximum(h + b2[...], 0.0)
        h = _bn(h, g2[...], be2[...])
        return h

    mu_n = mlp(mean_ref[...], mw1_ref, mb1_ref, mg1_ref, mbe1_ref,
               mw2_ref, mb2_ref, mg2_ref, mbe2_ref)
    sig_n = mlp(max_ref[...], sw1_ref, sb1_ref, sg1_ref, sbe1_ref,
                sw2_ref, sb2_ref, sg2_ref, sbe2_ref)

    b = batch_ref[...]
    seg = jax.lax.broadcasted_iota(jnp.int32, (NB, N), 0)
    onehot = (seg == b).astype(jnp.float32)
    cnt = jnp.sum(onehot, axis=1, keepdims=True)
    inv = 1.0 / jnp.maximum(cnt, 1.0)
    mu = jnp.dot(onehot, mu_n, preferred_element_type=jnp.float32,
                 precision=jax.lax.Precision.HIGHEST) * inv
    logvar = jnp.dot(onehot, sig_n, preferred_element_type=jnp.float32,
                     precision=jax.lax.Precision.HIGHEST) * inv
    z = mu + eps_ref[...] * jnp.exp(0.5 * logvar)
    h3 = jnp.maximum(
        jnp.dot(z, w3_ref[...], preferred_element_type=jnp.float32)
        + b3_ref[...], 0.0)
    rec = jnp.dot(h3, w4_ref[...], preferred_element_type=jnp.float32) \
        + b4_ref[...]
    rec_ref[...] = rec
    mu_ref[...] = mu
    logvar_ref[...] = logvar


def _bn_ref(h, g, be):
    m = jnp.mean(h, axis=0)
    v = jnp.var(h, axis=0)
    return g * (h - m) / jnp.sqrt(v + 1e-5) + be


def kernel(x, batch, params):
    p = params
    batch2 = batch.reshape(1, N)

    h = x
    for i in (1, 2):
        h = h @ p['fc1_W%d' % i] + p['fc1_b%d' % i]
        h = jax.nn.relu(h)
        h = _bn_ref(h, p['fc1_g%d' % i], p['fc1_be%d' % i])
    h1 = h

    idx = pl.pallas_call(
        _knn_idx_body,
        grid=(N // BLK,),
        in_specs=[
            pl.BlockSpec((N, 128), lambda i: (0, 0)),
            pl.BlockSpec((1, N), lambda i: (0, 0)),
        ],
        out_specs=pl.BlockSpec((BLK, K), lambda i: (i, 0)),
        out_shape=jax.ShapeDtypeStruct((N, K), jnp.int32),
        scratch_shapes=[pltpu.VMEM((BLK, N), jnp.float32)],
    )(h1, batch2)

    nbr_mean, nbr_max = _sc_gather_meanmax(h1, idx.reshape(N * K))

    eps = jax.random.normal(jax.random.key(1), (NB, 128), dtype=jnp.float32)

    rec, mu, logvar = pl.pallas_call(
        _head_body,
        out_shape=[
            jax.ShapeDtypeStruct((NB, 3 * 2048), jnp.float32),
            jax.ShapeDtypeStruct((NB, 128), jnp.float32),
            jax.ShapeDtypeStruct((NB, 128), jnp.float32),
        ],
    )(nbr_mean, nbr_max, batch2, eps,
      p['mu_W1'], p['mu_b1'][None, :], p['mu_g1'][None, :],
      p['mu_be1'][None, :], p['mu_W2'], p['mu_b2'][None, :],
      p['mu_g2'][None, :], p['mu_be2'][None, :],
      p['sig_W1'], p['sig_b1'][None, :], p['sig_g1'][None, :],
      p['sig_be1'][None, :], p['sig_W2'], p['sig_b2'][None, :],
      p['sig_g2'][None, :], p['sig_be2'][None, :],
      p['fc3_W'], p['fc3_b'][None, :], p['fc4_W'], p['fc4_b'][None, :])

    return (rec, mu, logvar)
